# baseline (speedup 1.0000x reference)
import functools

import jax
import jax.numpy as jnp
from jax.experimental import pallas as pl
from jax.experimental.pallas import tpu as pltpu


BLK = 64
CHUNK = 8


def _flash_body(counts_ref, order_ref, q_ref, k_ref, v_ref, o_ref,
                ks_ref, vs_ref, *, num_blocks, scale):
    i = pl.program_id(1)
    cnt = counts_ref[i]
    qb = q_ref[0]

    def chunk_step(c, carry):
        m, l, acc = carry
        base = c * CHUNK
        for j in range(CHUNK):
            idx = order_ref[i, base + j]
            ks_ref[pl.ds(j * BLK, BLK), :] = k_ref[0, pl.ds(idx * BLK, BLK), :]
            vs_ref[pl.ds(j * BLK, BLK), :] = v_ref[0, pl.ds(idx * BLK, BLK), :]
        s = jax.lax.dot_general(
            qb, ks_ref[...], (((1,), (1,)), ((), ())),
            preferred_element_type=jnp.float32) * scale
        col = jax.lax.broadcasted_iota(jnp.int32, (BLK, CHUNK * BLK), 1)
        valid = (base + col // BLK) < cnt
        s = jnp.where(valid, s, -1e30)
        m_new = jnp.maximum(m, jnp.max(s, axis=1, keepdims=True))
        p = jnp.exp(s - m_new)
        corr = jnp.exp(m - m_new)
        l_new = l * corr + jnp.sum(p, axis=1, keepdims=True)
        acc_new = acc * corr + jax.lax.dot_general(
            p, vs_ref[...], (((1,), (0,)), ((), ())),
            preferred_element_type=jnp.float32)
        return m_new, l_new, acc_new

    n_chunks = (cnt + CHUNK - 1) // CHUNK
    init = (jnp.full((BLK, 1), -1e30, jnp.float32),
            jnp.zeros((BLK, 1), jnp.float32),
            jnp.zeros((BLK, BLK), jnp.float32))
    m, l, acc = jax.lax.fori_loop(0, n_chunks, chunk_step, init)
    o_ref[0] = acc / l


def kernel(query_layer, key_layer, value_layer, attention_mask):
    b, h, s, d = query_layer.shape
    bh = b * h
    nb = s // BLK

    bm = attention_mask[::BLK, ::BLK]
    counts = jnp.sum(bm, axis=1).astype(jnp.int32)
    order = jnp.argsort(-bm, axis=1, stable=True).astype(jnp.int32)
    pad = (-nb) % CHUNK
    if pad:
        order = jnp.concatenate(
            [order, jnp.zeros((nb, pad), jnp.int32)], axis=1)

    q = query_layer.reshape(bh, s, d)
    k = key_layer.reshape(bh, s, d)
    v = value_layer.reshape(bh, s, d)

    grid = (bh, nb)
    out = pl.pallas_call(
        functools.partial(_flash_body, num_blocks=nb,
                          scale=1.0 / (d ** 0.5)),
        grid_spec=pltpu.PrefetchScalarGridSpec(
            num_scalar_prefetch=2,
            grid=grid,
            in_specs=[
                pl.BlockSpec((1, BLK, d), lambda g, i, *_: (g, i, 0)),
                pl.BlockSpec((1, s, d), lambda g, i, *_: (g, 0, 0)),
                pl.BlockSpec((1, s, d), lambda g, i, *_: (g, 0, 0)),
            ],
            out_specs=pl.BlockSpec((1, BLK, d), lambda g, i, *_: (g, i, 0)),
            scratch_shapes=[
                pltpu.VMEM((CHUNK * BLK, d), jnp.float32),
                pltpu.VMEM((CHUNK * BLK, d), jnp.float32),
            ],
        ),
        out_shape=jax.ShapeDtypeStruct((bh, s, d), jnp.float32),
    )(counts, order, q, k, v)
    return out.reshape(b, h, s, d)
